# Initial kernel scaffold; baseline (speedup 1.0000x reference)
import jax
import jax.numpy as jnp
from jax.experimental import pallas as pl


def kernel(x, top_features, edge_index, batch, W1, b1, W2, b2, fc_W, fc_b):
    raise NotImplementedError("write your pallas kernel here")


# speedup vs baseline: 26.9517x; 26.9517x over previous
import functools

import jax
import jax.numpy as jnp
from jax import lax
from jax.experimental import pallas as pl
from jax.experimental.pallas import tpu as pltpu
from jax.experimental.pallas import tpu_sc as plsc

N = 10000
E = 320000
NF = 128
TF = 4
HID = 128
OUT = 16
G = 64

NCORE = 2
NSUB = 16
NW = NCORE * NSUB

CH = 125
ECH = E // CH
ROWS_PER_TILE = ECH // NW
HALF_ROWS = ROWS_PER_TILE // 2
NODE_SLICE = N // NSUB

EPT = E // NW
DEG_PAD = 10240

ROWB = 1000
GRID = N // ROWB

_P = jax.lax.Precision.HIGHEST


def _sc_deg_body(dst_hbm, batch_hbm, pdeg0_hbm, pdeg1_hbm, counts_hbm,
                 idx_d, bidx, ones_v, zeros_v, accd, accc):
    c = lax.axis_index("c")
    s = lax.axis_index("s")
    w = c * NSUB + s

    def _fill(r, _):
        ones_v[pl.ds(r * 16, 16)] = jnp.full((16,), 1.0, jnp.float32)
        return _
    lax.fori_loop(0, EPT // 16, _fill, 0)
    for j in range(8):
        zeros_v[pl.ds(j * 16, 16)] = jnp.zeros((16,), jnp.float32)

    for i in range(5):
        pltpu.sync_copy(zeros_v, accd.at[pl.ds(s * 640 + i * 128, 128)])

    @pl.when(jnp.logical_and(c == 0, s == 0))
    def _():
        pltpu.sync_copy(zeros_v, accc)

    pltpu.sync_copy(dst_hbm.at[w, 0], idx_d)
    plsc.subcore_barrier()
    pltpu.sync_copy(ones_v, accd.at[idx_d], add=True)

    @pl.when(jnp.logical_and(c == 0, s == 0))
    def _():
        pltpu.sync_copy(batch_hbm, bidx)
        pltpu.sync_copy(ones_v, accc.at[bidx], add=True)

    plsc.subcore_barrier()

    @pl.when(s < 10)
    def _():
        @pl.when(c == 0)
        def _():
            pltpu.sync_copy(accd.at[pl.ds(s * 1024, 1024)],
                            pdeg0_hbm.at[pl.ds(s * 1024, 1024)])

        @pl.when(c == 1)
        def _():
            pltpu.sync_copy(accd.at[pl.ds(s * 1024, 1024)],
                            pdeg1_hbm.at[pl.ds(s * 1024, 1024)])

    @pl.when(jnp.logical_and(c == 0, s == 0))
    def _():
        pltpu.sync_copy(accc, counts_hbm)


_sc_deg = functools.partial(
    pl.kernel,
    out_type=(jax.ShapeDtypeStruct((DEG_PAD,), jnp.float32),
              jax.ShapeDtypeStruct((DEG_PAD,), jnp.float32),
              jax.ShapeDtypeStruct((128,), jnp.float32)),
    mesh=plsc.VectorSubcoreMesh(core_axis_name="c", subcore_axis_name="s"),
    scratch_types=[
        pltpu.VMEM((EPT,), jnp.int32),
        pltpu.VMEM((N,), jnp.int32),
        pltpu.VMEM((EPT,), jnp.float32),
        pltpu.VMEM((128,), jnp.float32),
        pltpu.VMEM_SHARED((DEG_PAD,), jnp.float32),
        pltpu.VMEM_SHARED((128,), jnp.float32),
    ],
)(_sc_deg_body)


def _sc_agg_body(y_hbm, src_hbm, dst_hbm, part_hbm,
                 idx_s, idx_d, rows0, rows1, acc, gsem0, gsem1):
    c = lax.axis_index("c")
    s = lax.axis_index("s")
    w = c * NSUB + s

    def _zrow(r, _):
        for j in range(HID // 16):
            rows0[r, pl.ds(j * 16, 16)] = jnp.zeros((16,), jnp.float32)
        return _
    lax.fori_loop(0, CH, _zrow, 0)
    for i in range(NODE_SLICE // CH):
        pltpu.sync_copy(rows0, acc.at[pl.ds(s * NODE_SLICE + i * CH, CH)])

    plsc.subcore_barrier()

    for half in range(2):
        base = w * ROWS_PER_TILE + half * HALF_ROWS
        pltpu.sync_copy(src_hbm.at[pl.ds(base, HALF_ROWS)], idx_s)
        pltpu.sync_copy(dst_hbm.at[pl.ds(base, HALF_ROWS)], idx_d)
        pltpu.async_copy(y_hbm.at[idx_s.at[0]], rows0, gsem0)
        pltpu.async_copy(y_hbm.at[idx_s.at[1]], rows1, gsem1)

        def _pair(cc, carry):
            for b in range(2):
                rows = rows0 if b == 0 else rows1
                gsem = gsem0 if b == 0 else gsem1
                ch = cc * 2 + b
                pltpu.make_async_copy(y_hbm.at[idx_s.at[ch]], rows, gsem).wait()
                pltpu.sync_copy(rows, acc.at[idx_d.at[ch]], add=True)

                @pl.when(cc < HALF_ROWS // 2 - 1)
                def _():
                    pltpu.async_copy(y_hbm.at[idx_s.at[ch + 2]], rows, gsem)
            return carry

        lax.fori_loop(0, HALF_ROWS // 2, _pair, 0)

    plsc.subcore_barrier()

    @pl.when(s < 10)
    def _():
        pltpu.sync_copy(acc.at[pl.ds(s * 1000, 1000)],
                        part_hbm.at[c, pl.ds(s * 1000, 1000)])


_sc_agg = functools.partial(
    pl.kernel,
    out_type=jax.ShapeDtypeStruct((NCORE, N, HID), jnp.float32),
    mesh=plsc.VectorSubcoreMesh(core_axis_name="c", subcore_axis_name="s"),
    scratch_types=[
        pltpu.VMEM((HALF_ROWS, CH), jnp.int32),
        pltpu.VMEM((HALF_ROWS, CH), jnp.int32),
        pltpu.VMEM((CH, HID), jnp.float32),
        pltpu.VMEM((CH, HID), jnp.float32),
        pltpu.VMEM_SHARED((N, HID), jnp.float32),
        pltpu.SemaphoreType.DMA,
        pltpu.SemaphoreType.DMA,
    ],
)(_sc_agg_body)


def _k1_body(x_ref, top_ref, w1a_ref, w1b_ref, p0_ref, p1_ref,
             y_ref, dinv_ref):
    deg = p0_ref[...] + p1_ref[...] + 1.0
    dinv = lax.rsqrt(deg)
    xw = (jnp.dot(x_ref[...], w1a_ref[...], precision=_P)
          + jnp.dot(top_ref[...], w1b_ref[...], precision=_P))
    y_ref[...] = xw * dinv
    dinv_ref[...] = dinv


def _k1(x, top, w1a, w1b, p0, p1):
    return pl.pallas_call(
        _k1_body,
        grid=(GRID,),
        in_specs=[
            pl.BlockSpec((ROWB, NF), lambda i: (i, 0)),
            pl.BlockSpec((ROWB, TF), lambda i: (i, 0)),
            pl.BlockSpec((NF, HID), lambda i: (0, 0)),
            pl.BlockSpec((TF, HID), lambda i: (0, 0)),
            pl.BlockSpec((ROWB, 1), lambda i: (i, 0)),
            pl.BlockSpec((ROWB, 1), lambda i: (i, 0)),
        ],
        out_specs=[
            pl.BlockSpec((ROWB, HID), lambda i: (i, 0)),
            pl.BlockSpec((ROWB, 1), lambda i: (i, 0)),
        ],
        out_shape=[
            jax.ShapeDtypeStruct((N, HID), jnp.float32),
            jax.ShapeDtypeStruct((N, 1), jnp.float32),
        ],
    )(x, top, w1a, w1b, p0, p1)


def _k2_body(pa_ref, pb_ref, y1_ref, dinv_ref, b1_ref, w2_ref, y2_ref):
    dinv = dinv_ref[...]
    h1 = jax.nn.relu((pa_ref[...] + pb_ref[...] + y1_ref[...]) * dinv
                     + b1_ref[...])
    y2_ref[...] = jnp.dot(h1, w2_ref[...], precision=_P) * dinv


def _k2(pa, pb, y1, dinv, b1, w2):
    return pl.pallas_call(
        _k2_body,
        grid=(GRID,),
        in_specs=[
            pl.BlockSpec((ROWB, HID), lambda i: (i, 0)),
            pl.BlockSpec((ROWB, HID), lambda i: (i, 0)),
            pl.BlockSpec((ROWB, HID), lambda i: (i, 0)),
            pl.BlockSpec((ROWB, 1), lambda i: (i, 0)),
            pl.BlockSpec((1, HID), lambda i: (0, 0)),
            pl.BlockSpec((HID, HID), lambda i: (0, 0)),
        ],
        out_specs=pl.BlockSpec((ROWB, HID), lambda i: (i, 0)),
        out_shape=jax.ShapeDtypeStruct((N, HID), jnp.float32),
    )(pa, pb, y1, dinv, b1, w2)


def _k3_body(pa_ref, pb_ref, y2_ref, dinv_ref, b2_ref, batch_ref,
             counts_ref, fcw_ref, fcb_ref, out_ref, pooled):
    i = pl.program_id(0)

    @pl.when(i == 0)
    def _():
        pooled[...] = jnp.zeros((G, HID), jnp.float32)

    h2 = jax.nn.relu((pa_ref[...] + pb_ref[...] + y2_ref[...])
                     * dinv_ref[...] + b2_ref[...])
    bb = batch_ref[0]
    gid = lax.broadcasted_iota(jnp.int32, (G, ROWB), 0)
    mask = (bb == gid).astype(jnp.float32)
    pooled[...] += jnp.dot(mask, h2, precision=_P)

    @pl.when(i == GRID - 1)
    def _():
        cnt = jnp.maximum(counts_ref[...], 1.0)
        out_ref[...] = (jnp.dot(pooled[...] / cnt, fcw_ref[...],
                                precision=_P) + fcb_ref[...])


def _k3(pa, pb, y2, dinv, b2, batch3, counts, fcw, fcb):
    return pl.pallas_call(
        _k3_body,
        grid=(GRID,),
        in_specs=[
            pl.BlockSpec((ROWB, HID), lambda i: (i, 0)),
            pl.BlockSpec((ROWB, HID), lambda i: (i, 0)),
            pl.BlockSpec((ROWB, HID), lambda i: (i, 0)),
            pl.BlockSpec((ROWB, 1), lambda i: (i, 0)),
            pl.BlockSpec((1, HID), lambda i: (0, 0)),
            pl.BlockSpec((1, 1, ROWB), lambda i: (i, 0, 0)),
            pl.BlockSpec((G, 1), lambda i: (0, 0)),
            pl.BlockSpec((HID, OUT), lambda i: (0, 0)),
            pl.BlockSpec((1, OUT), lambda i: (0, 0)),
        ],
        out_specs=pl.BlockSpec((G, OUT), lambda i: (0, 0)),
        out_shape=jax.ShapeDtypeStruct((G, OUT), jnp.float32),
        scratch_shapes=[pltpu.VMEM((G, HID), jnp.float32)],
    )(pa, pb, y2, dinv, b2, batch3, counts, fcw, fcb)


def kernel(x, top_features, edge_index, batch, W1, b1, W2, b2, fc_W, fc_b):
    src2d = edge_index[0].reshape(ECH, CH)
    dst2d = edge_index[1].reshape(ECH, CH)
    pdeg0, pdeg1, counts = _sc_deg(edge_index[1].reshape(NW, 1, EPT), batch)
    p0 = pdeg0[:N].reshape(N, 1)
    p1 = pdeg1[:N].reshape(N, 1)

    y1, dinv = _k1(x, top_features, W1[:NF], W1[NF:], p0, p1)

    part1 = _sc_agg(y1, src2d, dst2d)
    y2 = _k2(part1[0], part1[1], y1, dinv, b1.reshape(1, HID), W2)

    part2 = _sc_agg(y2, src2d, dst2d)
    out = _k3(part2[0], part2[1], y2, dinv, b2.reshape(1, HID),
              batch.reshape(GRID, 1, ROWB), counts[:G].reshape(G, 1),
              fc_W, fc_b.reshape(1, OUT))
    return out


# speedup vs baseline: 26.9639x; 1.0005x over previous
import functools

import jax
import jax.numpy as jnp
from jax import lax
from jax.experimental import pallas as pl
from jax.experimental.pallas import tpu as pltpu
from jax.experimental.pallas import tpu_sc as plsc

N = 10000
E = 320000
NF = 128
TF = 4
HID = 128
OUT = 16
G = 64

NCORE = 2
NSUB = 16
NW = NCORE * NSUB

CH = 125
ECH = E // CH
ROWS_PER_TILE = ECH // NW
HALF_ROWS = ROWS_PER_TILE // 2
NODE_SLICE = N // NSUB

EPT = E // NW
DEG_PAD = 10240

ROWB = 1000
GRID = N // ROWB

_P = jax.lax.Precision.HIGHEST


def _sc_deg_body(dst_hbm, batch_hbm, pdeg0_hbm, pdeg1_hbm, counts_hbm,
                 idx_d, bidx, ones_v, zeros_v, accd, accc):
    c = lax.axis_index("c")
    s = lax.axis_index("s")
    w = c * NSUB + s

    def _fill(r, _):
        ones_v[pl.ds(r * 16, 16)] = jnp.full((16,), 1.0, jnp.float32)
        return _
    lax.fori_loop(0, EPT // 16, _fill, 0)
    for j in range(8):
        zeros_v[pl.ds(j * 16, 16)] = jnp.zeros((16,), jnp.float32)

    for i in range(5):
        pltpu.sync_copy(zeros_v, accd.at[pl.ds(s * 640 + i * 128, 128)])

    @pl.when(jnp.logical_and(c == 0, s == 0))
    def _():
        pltpu.sync_copy(zeros_v, accc)

    pltpu.sync_copy(dst_hbm.at[w, 0], idx_d)
    plsc.subcore_barrier()
    pltpu.sync_copy(ones_v, accd.at[idx_d], add=True)

    @pl.when(jnp.logical_and(c == 0, s == 0))
    def _():
        pltpu.sync_copy(batch_hbm, bidx)
        pltpu.sync_copy(ones_v, accc.at[bidx], add=True)

    plsc.subcore_barrier()

    @pl.when(s < 10)
    def _():
        @pl.when(c == 0)
        def _():
            pltpu.sync_copy(accd.at[pl.ds(s * 1024, 1024)],
                            pdeg0_hbm.at[pl.ds(s * 1024, 1024)])

        @pl.when(c == 1)
        def _():
            pltpu.sync_copy(accd.at[pl.ds(s * 1024, 1024)],
                            pdeg1_hbm.at[pl.ds(s * 1024, 1024)])

    @pl.when(jnp.logical_and(c == 0, s == 0))
    def _():
        pltpu.sync_copy(accc, counts_hbm)


_sc_deg = functools.partial(
    pl.kernel,
    out_type=(jax.ShapeDtypeStruct((DEG_PAD,), jnp.float32),
              jax.ShapeDtypeStruct((DEG_PAD,), jnp.float32),
              jax.ShapeDtypeStruct((128,), jnp.float32)),
    mesh=plsc.VectorSubcoreMesh(core_axis_name="c", subcore_axis_name="s"),
    scratch_types=[
        pltpu.VMEM((EPT,), jnp.int32),
        pltpu.VMEM((N,), jnp.int32),
        pltpu.VMEM((EPT,), jnp.float32),
        pltpu.VMEM((128,), jnp.float32),
        pltpu.VMEM_SHARED((DEG_PAD,), jnp.float32),
        pltpu.VMEM_SHARED((128,), jnp.float32),
    ],
)(_sc_deg_body)


def _sc_agg_body(y_hbm, src_hbm, dst_hbm, part_hbm,
                 idx_s, idx_d, rows0, rows1, acc,
                 gsem0, gsem1, ssem0, ssem1):
    c = lax.axis_index("c")
    s = lax.axis_index("s")
    w = c * NSUB + s

    def _zrow(r, _):
        for j in range(HID // 16):
            rows0[r, pl.ds(j * 16, 16)] = jnp.zeros((16,), jnp.float32)
        return _
    lax.fori_loop(0, CH, _zrow, 0)
    for i in range(NODE_SLICE // CH):
        pltpu.sync_copy(rows0, acc.at[pl.ds(s * NODE_SLICE + i * CH, CH)])

    plsc.subcore_barrier()

    for half in range(2):
        base = w * ROWS_PER_TILE + half * HALF_ROWS
        pltpu.sync_copy(src_hbm.at[pl.ds(base, HALF_ROWS)], idx_s)
        pltpu.sync_copy(dst_hbm.at[pl.ds(base, HALF_ROWS)], idx_d)
        pltpu.async_copy(y_hbm.at[idx_s.at[0]], rows0, gsem0)

        def _pair(cc, carry):
            for b in range(2):
                rows = rows0 if b == 0 else rows1
                gsem = gsem0 if b == 0 else gsem1
                ssem = ssem0 if b == 0 else ssem1
                orows = rows1 if b == 0 else rows0
                ogsem = gsem1 if b == 0 else gsem0
                ossem = ssem1 if b == 0 else ssem0
                ch = cc * 2 + b

                @pl.when(ch + 1 < HALF_ROWS)
                def _():
                    @pl.when(ch >= 1)
                    def _():
                        pltpu.make_async_copy(
                            orows, acc.at[idx_d.at[ch - 1]], ossem).wait()
                    pltpu.async_copy(y_hbm.at[idx_s.at[ch + 1]], orows, ogsem)

                pltpu.make_async_copy(y_hbm.at[idx_s.at[ch]], rows, gsem).wait()
                pltpu.async_copy(rows, acc.at[idx_d.at[ch]], ssem, add=True)
            return carry

        lax.fori_loop(0, HALF_ROWS // 2, _pair, 0)
        pltpu.make_async_copy(rows0, acc.at[idx_d.at[HALF_ROWS - 2]],
                              ssem0).wait()
        pltpu.make_async_copy(rows1, acc.at[idx_d.at[HALF_ROWS - 1]],
                              ssem1).wait()

    plsc.subcore_barrier()

    @pl.when(s < 10)
    def _():
        pltpu.sync_copy(acc.at[pl.ds(s * 1000, 1000)],
                        part_hbm.at[c, pl.ds(s * 1000, 1000)])


_sc_agg = functools.partial(
    pl.kernel,
    out_type=jax.ShapeDtypeStruct((NCORE, N, HID), jnp.float32),
    mesh=plsc.VectorSubcoreMesh(core_axis_name="c", subcore_axis_name="s"),
    scratch_types=[
        pltpu.VMEM((HALF_ROWS, CH), jnp.int32),
        pltpu.VMEM((HALF_ROWS, CH), jnp.int32),
        pltpu.VMEM((CH, HID), jnp.float32),
        pltpu.VMEM((CH, HID), jnp.float32),
        pltpu.VMEM_SHARED((N, HID), jnp.float32),
        pltpu.SemaphoreType.DMA,
        pltpu.SemaphoreType.DMA,
        pltpu.SemaphoreType.DMA,
        pltpu.SemaphoreType.DMA,
    ],
)(_sc_agg_body)


def _k1_body(x_ref, top_ref, w1a_ref, w1b_ref, p0_ref, p1_ref,
             y_ref, dinv_ref):
    deg = p0_ref[...] + p1_ref[...] + 1.0
    dinv = lax.rsqrt(deg)
    xw = (jnp.dot(x_ref[...], w1a_ref[...], precision=_P)
          + jnp.dot(top_ref[...], w1b_ref[...], precision=_P))
    y_ref[...] = xw * dinv
    dinv_ref[...] = dinv


def _k1(x, top, w1a, w1b, p0, p1):
    return pl.pallas_call(
        _k1_body,
        grid=(GRID,),
        in_specs=[
            pl.BlockSpec((ROWB, NF), lambda i: (i, 0)),
            pl.BlockSpec((ROWB, TF), lambda i: (i, 0)),
            pl.BlockSpec((NF, HID), lambda i: (0, 0)),
            pl.BlockSpec((TF, HID), lambda i: (0, 0)),
            pl.BlockSpec((ROWB, 1), lambda i: (i, 0)),
            pl.BlockSpec((ROWB, 1), lambda i: (i, 0)),
        ],
        out_specs=[
            pl.BlockSpec((ROWB, HID), lambda i: (i, 0)),
            pl.BlockSpec((ROWB, 1), lambda i: (i, 0)),
        ],
        out_shape=[
            jax.ShapeDtypeStruct((N, HID), jnp.float32),
            jax.ShapeDtypeStruct((N, 1), jnp.float32),
        ],
    )(x, top, w1a, w1b, p0, p1)


def _k2_body(pa_ref, pb_ref, y1_ref, dinv_ref, b1_ref, w2_ref, y2_ref):
    dinv = dinv_ref[...]
    h1 = jax.nn.relu((pa_ref[...] + pb_ref[...] + y1_ref[...]) * dinv
                     + b1_ref[...])
    y2_ref[...] = jnp.dot(h1, w2_ref[...], precision=_P) * dinv


def _k2(pa, pb, y1, dinv, b1, w2):
    return pl.pallas_call(
        _k2_body,
        grid=(GRID,),
        in_specs=[
            pl.BlockSpec((ROWB, HID), lambda i: (i, 0)),
            pl.BlockSpec((ROWB, HID), lambda i: (i, 0)),
            pl.BlockSpec((ROWB, HID), lambda i: (i, 0)),
            pl.BlockSpec((ROWB, 1), lambda i: (i, 0)),
            pl.BlockSpec((1, HID), lambda i: (0, 0)),
            pl.BlockSpec((HID, HID), lambda i: (0, 0)),
        ],
        out_specs=pl.BlockSpec((ROWB, HID), lambda i: (i, 0)),
        out_shape=jax.ShapeDtypeStruct((N, HID), jnp.float32),
    )(pa, pb, y1, dinv, b1, w2)


def _k3_body(pa_ref, pb_ref, y2_ref, dinv_ref, b2_ref, batch_ref,
             counts_ref, fcw_ref, fcb_ref, out_ref, pooled):
    i = pl.program_id(0)

    @pl.when(i == 0)
    def _():
        pooled[...] = jnp.zeros((G, HID), jnp.float32)

    h2 = jax.nn.relu((pa_ref[...] + pb_ref[...] + y2_ref[...])
                     * dinv_ref[...] + b2_ref[...])
    bb = batch_ref[0]
    gid = lax.broadcasted_iota(jnp.int32, (G, ROWB), 0)
    mask = (bb == gid).astype(jnp.float32)
    pooled[...] += jnp.dot(mask, h2, precision=_P)

    @pl.when(i == GRID - 1)
    def _():
        cnt = jnp.maximum(counts_ref[...], 1.0)
        out_ref[...] = (jnp.dot(pooled[...] / cnt, fcw_ref[...],
                                precision=_P) + fcb_ref[...])


def _k3(pa, pb, y2, dinv, b2, batch3, counts, fcw, fcb):
    return pl.pallas_call(
        _k3_body,
        grid=(GRID,),
        in_specs=[
            pl.BlockSpec((ROWB, HID), lambda i: (i, 0)),
            pl.BlockSpec((ROWB, HID), lambda i: (i, 0)),
            pl.BlockSpec((ROWB, HID), lambda i: (i, 0)),
            pl.BlockSpec((ROWB, 1), lambda i: (i, 0)),
            pl.BlockSpec((1, HID), lambda i: (0, 0)),
            pl.BlockSpec((1, 1, ROWB), lambda i: (i, 0, 0)),
            pl.BlockSpec((G, 1), lambda i: (0, 0)),
            pl.BlockSpec((HID, OUT), lambda i: (0, 0)),
            pl.BlockSpec((1, OUT), lambda i: (0, 0)),
        ],
        out_specs=pl.BlockSpec((G, OUT), lambda i: (0, 0)),
        out_shape=jax.ShapeDtypeStruct((G, OUT), jnp.float32),
        scratch_shapes=[pltpu.VMEM((G, HID), jnp.float32)],
    )(pa, pb, y2, dinv, b2, batch3, counts, fcw, fcb)


def kernel(x, top_features, edge_index, batch, W1, b1, W2, b2, fc_W, fc_b):
    src2d = edge_index[0].reshape(ECH, CH)
    dst2d = edge_index[1].reshape(ECH, CH)
    pdeg0, pdeg1, counts = _sc_deg(edge_index[1].reshape(NW, 1, EPT), batch)
    p0 = pdeg0[:N].reshape(N, 1)
    p1 = pdeg1[:N].reshape(N, 1)

    y1, dinv = _k1(x, top_features, W1[:NF], W1[NF:], p0, p1)

    part1 = _sc_agg(y1, src2d, dst2d)
    y2 = _k2(part1[0], part1[1], y1, dinv, b1.reshape(1, HID), W2)

    part2 = _sc_agg(y2, src2d, dst2d)
    out = _k3(part2[0], part2[1], y2, dinv, b2.reshape(1, HID),
              batch.reshape(GRID, 1, ROWB), counts[:G].reshape(G, 1),
              fc_W, fc_b.reshape(1, OUT))
    return out


# speedup vs baseline: 26.9640x; 1.0000x over previous
import functools

import jax
import jax.numpy as jnp
from jax import lax
from jax.experimental import pallas as pl
from jax.experimental.pallas import tpu as pltpu
from jax.experimental.pallas import tpu_sc as plsc

N = 10000
E = 320000
NF = 128
TF = 4
HID = 128
OUT = 16
G = 64

NCORE = 2
NSUB = 16
NW = NCORE * NSUB

CH = 125
ECH = E // CH
ROWS_PER_TILE = ECH // NW
HALF_ROWS = ROWS_PER_TILE // 2
NODE_SLICE = N // NSUB
ZROWS = 25

EPT = E // NW
DEG_PAD = 10240

ROWB = 1000
GRID = N // ROWB

_P = jax.lax.Precision.HIGHEST


def _sc_deg_body(dst_hbm, batch_hbm, pdeg0_hbm, pdeg1_hbm, counts_hbm,
                 idx_d, bidx, ones_v, zeros_v, accd, accc):
    c = lax.axis_index("c")
    s = lax.axis_index("s")
    w = c * NSUB + s

    def _fill(r, _):
        ones_v[pl.ds(r * 16, 16)] = jnp.full((16,), 1.0, jnp.float32)
        return _
    lax.fori_loop(0, EPT // 16, _fill, 0)
    for j in range(8):
        zeros_v[pl.ds(j * 16, 16)] = jnp.zeros((16,), jnp.float32)

    for i in range(5):
        pltpu.sync_copy(zeros_v, accd.at[pl.ds(s * 640 + i * 128, 128)])

    @pl.when(jnp.logical_and(c == 0, s == 0))
    def _():
        pltpu.sync_copy(zeros_v, accc)

    pltpu.sync_copy(dst_hbm.at[w, 0], idx_d)
    plsc.subcore_barrier()
    pltpu.sync_copy(ones_v, accd.at[idx_d], add=True)

    @pl.when(jnp.logical_and(c == 0, s == 0))
    def _():
        pltpu.sync_copy(batch_hbm, bidx)
        pltpu.sync_copy(ones_v, accc.at[bidx], add=True)

    plsc.subcore_barrier()

    @pl.when(s < 10)
    def _():
        @pl.when(c == 0)
        def _():
            pltpu.sync_copy(accd.at[pl.ds(s * 1024, 1024)],
                            pdeg0_hbm.at[pl.ds(s * 1024, 1024)])

        @pl.when(c == 1)
        def _():
            pltpu.sync_copy(accd.at[pl.ds(s * 1024, 1024)],
                            pdeg1_hbm.at[pl.ds(s * 1024, 1024)])

    @pl.when(jnp.logical_and(c == 0, s == 0))
    def _():
        pltpu.sync_copy(accc, counts_hbm)


_sc_deg = functools.partial(
    pl.kernel,
    out_type=(jax.ShapeDtypeStruct((DEG_PAD,), jnp.float32),
              jax.ShapeDtypeStruct((DEG_PAD,), jnp.float32),
              jax.ShapeDtypeStruct((128,), jnp.float32)),
    mesh=plsc.VectorSubcoreMesh(core_axis_name="c", subcore_axis_name="s"),
    scratch_types=[
        pltpu.VMEM((EPT,), jnp.int32),
        pltpu.VMEM((N,), jnp.int32),
        pltpu.VMEM((EPT,), jnp.float32),
        pltpu.VMEM((128,), jnp.float32),
        pltpu.VMEM_SHARED((DEG_PAD,), jnp.float32),
        pltpu.VMEM_SHARED((128,), jnp.float32),
    ],
)(_sc_deg_body)


def _sc_agg_body(y_hbm, src_hbm, dst_hbm, part_hbm,
                 idx_s, idx_d, rows0, rows1, zbuf, acc,
                 gsem0, gsem1, ssem0, ssem1):
    c = lax.axis_index("c")
    s = lax.axis_index("s")
    w = c * NSUB + s

    for half in range(2):
        base = w * ROWS_PER_TILE + half * HALF_ROWS
        pltpu.sync_copy(src_hbm.at[pl.ds(base, HALF_ROWS)], idx_s)
        pltpu.sync_copy(dst_hbm.at[pl.ds(base, HALF_ROWS)], idx_d)
        pltpu.async_copy(y_hbm.at[idx_s.at[0]], rows0, gsem0)

        if half == 0:
            def _zrow(r, _):
                for j in range(HID // 16):
                    zbuf[r, pl.ds(j * 16, 16)] = jnp.zeros((16,), jnp.float32)
                return _
            lax.fori_loop(0, ZROWS, _zrow, 0)
            for i in range(NODE_SLICE // ZROWS):
                pltpu.sync_copy(
                    zbuf, acc.at[pl.ds(s * NODE_SLICE + i * ZROWS, ZROWS)])
            plsc.subcore_barrier()

        def _pair(cc, carry):
            for b in range(2):
                rows = rows0 if b == 0 else rows1
                gsem = gsem0 if b == 0 else gsem1
                ssem = ssem0 if b == 0 else ssem1
                orows = rows1 if b == 0 else rows0
                ogsem = gsem1 if b == 0 else gsem0
                ossem = ssem1 if b == 0 else ssem0
                ch = cc * 2 + b

                @pl.when(ch + 1 < HALF_ROWS)
                def _():
                    @pl.when(ch >= 1)
                    def _():
                        pltpu.make_async_copy(
                            orows, acc.at[idx_d.at[ch - 1]], ossem).wait()
                    pltpu.async_copy(y_hbm.at[idx_s.at[ch + 1]], orows, ogsem)

                pltpu.make_async_copy(y_hbm.at[idx_s.at[ch]], rows, gsem).wait()
                pltpu.async_copy(rows, acc.at[idx_d.at[ch]], ssem, add=True)
            return carry

        lax.fori_loop(0, HALF_ROWS // 2, _pair, 0)
        pltpu.make_async_copy(rows0, acc.at[idx_d.at[HALF_ROWS - 2]],
                              ssem0).wait()
        pltpu.make_async_copy(rows1, acc.at[idx_d.at[HALF_ROWS - 1]],
                              ssem1).wait()

    plsc.subcore_barrier()

    @pl.when(s < 15)
    def _():
        pltpu.sync_copy(acc.at[pl.ds(s * 624, 624)],
                        part_hbm.at[c, pl.ds(s * 624, 624)])

    @pl.when(s == 15)
    def _():
        pltpu.sync_copy(acc.at[pl.ds(15 * 624, 640)],
                        part_hbm.at[c, pl.ds(15 * 624, 640)])


_sc_agg = functools.partial(
    pl.kernel,
    out_type=jax.ShapeDtypeStruct((NCORE, N, HID), jnp.float32),
    mesh=plsc.VectorSubcoreMesh(core_axis_name="c", subcore_axis_name="s"),
    scratch_types=[
        pltpu.VMEM((HALF_ROWS, CH), jnp.int32),
        pltpu.VMEM((HALF_ROWS, CH), jnp.int32),
        pltpu.VMEM((CH, HID), jnp.float32),
        pltpu.VMEM((CH, HID), jnp.float32),
        pltpu.VMEM((ZROWS, HID), jnp.float32),
        pltpu.VMEM_SHARED((N, HID), jnp.float32),
        pltpu.SemaphoreType.DMA,
        pltpu.SemaphoreType.DMA,
        pltpu.SemaphoreType.DMA,
        pltpu.SemaphoreType.DMA,
    ],
)(_sc_agg_body)


def _k1a_body(x_ref, top_ref, w1a_ref, w1b_ref, xw_ref):
    xw_ref[...] = (jnp.dot(x_ref[...], w1a_ref[...], precision=_P)
                   + jnp.dot(top_ref[...], w1b_ref[...], precision=_P))


def _k1a(x, top, w1a, w1b):
    return pl.pallas_call(
        _k1a_body,
        grid=(GRID,),
        in_specs=[
            pl.BlockSpec((ROWB, NF), lambda i: (i, 0)),
            pl.BlockSpec((ROWB, TF), lambda i: (i, 0)),
            pl.BlockSpec((NF, HID), lambda i: (0, 0)),
            pl.BlockSpec((TF, HID), lambda i: (0, 0)),
        ],
        out_specs=pl.BlockSpec((ROWB, HID), lambda i: (i, 0)),
        out_shape=jax.ShapeDtypeStruct((N, HID), jnp.float32),
    )(x, top, w1a, w1b)


def _k1b_body(xw_ref, p0_ref, p1_ref, y_ref, dinv_ref):
    deg = p0_ref[...] + p1_ref[...] + 1.0
    dinv = lax.rsqrt(deg)
    y_ref[...] = xw_ref[...] * dinv
    dinv_ref[...] = dinv


def _k1b(xw, p0, p1):
    return pl.pallas_call(
        _k1b_body,
        grid=(GRID,),
        in_specs=[
            pl.BlockSpec((ROWB, HID), lambda i: (i, 0)),
            pl.BlockSpec((ROWB, 1), lambda i: (i, 0)),
            pl.BlockSpec((ROWB, 1), lambda i: (i, 0)),
        ],
        out_specs=[
            pl.BlockSpec((ROWB, HID), lambda i: (i, 0)),
            pl.BlockSpec((ROWB, 1), lambda i: (i, 0)),
        ],
        out_shape=[
            jax.ShapeDtypeStruct((N, HID), jnp.float32),
            jax.ShapeDtypeStruct((N, 1), jnp.float32),
        ],
    )(xw, p0, p1)


def _k2_body(pa_ref, pb_ref, y1_ref, dinv_ref, b1_ref, w2_ref, y2_ref):
    dinv = dinv_ref[...]
    h1 = jax.nn.relu((pa_ref[...] + pb_ref[...] + y1_ref[...]) * dinv
                     + b1_ref[...])
    y2_ref[...] = jnp.dot(h1, w2_ref[...], precision=_P) * dinv


def _k2(pa, pb, y1, dinv, b1, w2):
    return pl.pallas_call(
        _k2_body,
        grid=(GRID,),
        in_specs=[
            pl.BlockSpec((ROWB, HID), lambda i: (i, 0)),
            pl.BlockSpec((ROWB, HID), lambda i: (i, 0)),
            pl.BlockSpec((ROWB, HID), lambda i: (i, 0)),
            pl.BlockSpec((ROWB, 1), lambda i: (i, 0)),
            pl.BlockSpec((1, HID), lambda i: (0, 0)),
            pl.BlockSpec((HID, HID), lambda i: (0, 0)),
        ],
        out_specs=pl.BlockSpec((ROWB, HID), lambda i: (i, 0)),
        out_shape=jax.ShapeDtypeStruct((N, HID), jnp.float32),
    )(pa, pb, y1, dinv, b1, w2)


def _k3_body(pa_ref, pb_ref, y2_ref, dinv_ref, b2_ref, batch_ref,
             counts_ref, fcw_ref, fcb_ref, out_ref, pooled):
    i = pl.program_id(0)

    @pl.when(i == 0)
    def _():
        pooled[...] = jnp.zeros((G, HID), jnp.float32)

    h2 = jax.nn.relu((pa_ref[...] + pb_ref[...] + y2_ref[...])
                     * dinv_ref[...] + b2_ref[...])
    bb = batch_ref[0]
    gid = lax.broadcasted_iota(jnp.int32, (G, ROWB), 0)
    mask = (bb == gid).astype(jnp.float32)
    pooled[...] += jnp.dot(mask, h2, precision=_P)

    @pl.when(i == GRID - 1)
    def _():
        cnt = jnp.maximum(counts_ref[...], 1.0)
        out_ref[...] = (jnp.dot(pooled[...] / cnt, fcw_ref[...],
                                precision=_P) + fcb_ref[...])


def _k3(pa, pb, y2, dinv, b2, batch3, counts, fcw, fcb):
    return pl.pallas_call(
        _k3_body,
        grid=(GRID,),
        in_specs=[
            pl.BlockSpec((ROWB, HID), lambda i: (i, 0)),
            pl.BlockSpec((ROWB, HID), lambda i: (i, 0)),
            pl.BlockSpec((ROWB, HID), lambda i: (i, 0)),
            pl.BlockSpec((ROWB, 1), lambda i: (i, 0)),
            pl.BlockSpec((1, HID), lambda i: (0, 0)),
            pl.BlockSpec((1, 1, ROWB), lambda i: (i, 0, 0)),
            pl.BlockSpec((G, 1), lambda i: (0, 0)),
            pl.BlockSpec((HID, OUT), lambda i: (0, 0)),
            pl.BlockSpec((1, OUT), lambda i: (0, 0)),
        ],
        out_specs=pl.BlockSpec((G, OUT), lambda i: (0, 0)),
        out_shape=jax.ShapeDtypeStruct((G, OUT), jnp.float32),
        scratch_shapes=[pltpu.VMEM((G, HID), jnp.float32)],
    )(pa, pb, y2, dinv, b2, batch3, counts, fcw, fcb)


def kernel(x, top_features, edge_index, batch, W1, b1, W2, b2, fc_W, fc_b):
    src2d = edge_index[0].reshape(ECH, CH)
    dst2d = edge_index[1].reshape(ECH, CH)
    pdeg0, pdeg1, counts = _sc_deg(edge_index[1].reshape(NW, 1, EPT), batch)
    p0 = pdeg0[:N].reshape(N, 1)
    p1 = pdeg1[:N].reshape(N, 1)

    xw1 = _k1a(x, top_features, W1[:NF], W1[NF:])
    y1, dinv = _k1b(xw1, p0, p1)

    part1 = _sc_agg(y1, src2d, dst2d)
    y2 = _k2(part1[0], part1[1], y1, dinv, b1.reshape(1, HID), W2)

    part2 = _sc_agg(y2, src2d, dst2d)
    out = _k3(part2[0], part2[1], y2, dinv, b2.reshape(1, HID),
              batch.reshape(GRID, 1, ROWB), counts[:G].reshape(G, 1),
              fc_W, fc_b.reshape(1, OUT))
    return out


# speedup vs baseline: 27.0379x; 1.0027x over previous
import functools

import jax
import jax.numpy as jnp
from jax import lax
from jax.experimental import pallas as pl
from jax.experimental.pallas import tpu as pltpu
from jax.experimental.pallas import tpu_sc as plsc

N = 10000
E = 320000
NF = 128
TF = 4
HID = 128
OUT = 16
G = 64

NCORE = 2
NSUB = 16
NW = NCORE * NSUB

CH = 125
ECH = E // CH
ROWS_PER_TILE = ECH // NW
HALF_ROWS = ROWS_PER_TILE // 2
NODE_SLICE = N // NSUB
ZROWS = 25

EPT = E // NW
DEG_PAD = 10240

ROWB = 1000
GRID = N // ROWB

_P = jax.lax.Precision.HIGHEST


def _sc_deg_body(dst_hbm, batch_hbm, pdeg0_hbm, pdeg1_hbm, counts_hbm,
                 idx_d, bidx, ones_v, zeros_v, accd, accc, stsem):
    c = lax.axis_index("c")
    s = lax.axis_index("s")
    w = c * NSUB + s

    pltpu.async_copy(dst_hbm.at[w, 0], idx_d, stsem)

    @pl.when(c == 0)
    def _():
        pltpu.sync_copy(batch_hbm.at[s, 0], bidx)

    def _fill(r, _):
        ones_v[pl.ds(r * 16, 16)] = jnp.full((16,), 1.0, jnp.float32)
        return _
    lax.fori_loop(0, EPT // 16, _fill, 0)
    for j in range(8):
        zeros_v[pl.ds(j * 16, 16)] = jnp.zeros((16,), jnp.float32)

    for i in range(5):
        pltpu.sync_copy(zeros_v, accd.at[pl.ds(s * 640 + i * 128, 128)])

    @pl.when(jnp.logical_and(c == 0, s == 0))
    def _():
        pltpu.sync_copy(zeros_v, accc)

    pltpu.make_async_copy(dst_hbm.at[w, 0], idx_d, stsem).wait()
    plsc.subcore_barrier()

    pltpu.sync_copy(ones_v, accd.at[idx_d], add=True)

    @pl.when(c == 0)
    def _():
        pltpu.sync_copy(ones_v.at[pl.ds(0, N // NSUB)], accc.at[bidx],
                        add=True)

    plsc.subcore_barrier()

    @pl.when(s < 10)
    def _():
        @pl.when(c == 0)
        def _():
            pltpu.sync_copy(accd.at[pl.ds(s * 1024, 1024)],
                            pdeg0_hbm.at[pl.ds(s * 1024, 1024)])

        @pl.when(c == 1)
        def _():
            pltpu.sync_copy(accd.at[pl.ds(s * 1024, 1024)],
                            pdeg1_hbm.at[pl.ds(s * 1024, 1024)])

    @pl.when(jnp.logical_and(c == 0, s == 0))
    def _():
        pltpu.sync_copy(accc, counts_hbm)


_sc_deg = functools.partial(
    pl.kernel,
    out_type=(jax.ShapeDtypeStruct((DEG_PAD,), jnp.float32),
              jax.ShapeDtypeStruct((DEG_PAD,), jnp.float32),
              jax.ShapeDtypeStruct((128,), jnp.float32)),
    mesh=plsc.VectorSubcoreMesh(core_axis_name="c", subcore_axis_name="s"),
    scratch_types=[
        pltpu.VMEM((EPT,), jnp.int32),
        pltpu.VMEM((N // NSUB,), jnp.int32),
        pltpu.VMEM((EPT,), jnp.float32),
        pltpu.VMEM((128,), jnp.float32),
        pltpu.VMEM_SHARED((DEG_PAD,), jnp.float32),
        pltpu.VMEM_SHARED((128,), jnp.float32),
        pltpu.SemaphoreType.DMA,
    ],
)(_sc_deg_body)


def _sc_agg_body(y_hbm, src_hbm, dst_hbm, part_hbm,
                 idx_s, idx_d, rows0, rows1, zbuf, acc,
                 gsem0, gsem1, ssem0, ssem1):
    c = lax.axis_index("c")
    s = lax.axis_index("s")
    w = c * NSUB + s

    for half in range(2):
        base = w * ROWS_PER_TILE + half * HALF_ROWS
        pltpu.sync_copy(src_hbm.at[pl.ds(base, HALF_ROWS)], idx_s)
        pltpu.sync_copy(dst_hbm.at[pl.ds(base, HALF_ROWS)], idx_d)
        pltpu.async_copy(y_hbm.at[idx_s.at[0]], rows0, gsem0)

        if half == 0:
            def _zrow(r, _):
                for j in range(HID // 16):
                    zbuf[r, pl.ds(j * 16, 16)] = jnp.zeros((16,), jnp.float32)
                return _
            lax.fori_loop(0, ZROWS, _zrow, 0)
            for i in range(NODE_SLICE // ZROWS):
                pltpu.sync_copy(
                    zbuf, acc.at[pl.ds(s * NODE_SLICE + i * ZROWS, ZROWS)])
            plsc.subcore_barrier()

        def _pair(cc, carry):
            for b in range(2):
                rows = rows0 if b == 0 else rows1
                gsem = gsem0 if b == 0 else gsem1
                ssem = ssem0 if b == 0 else ssem1
                orows = rows1 if b == 0 else rows0
                ogsem = gsem1 if b == 0 else gsem0
                ossem = ssem1 if b == 0 else ssem0
                ch = cc * 2 + b

                @pl.when(ch + 1 < HALF_ROWS)
                def _():
                    @pl.when(ch >= 1)
                    def _():
                        pltpu.make_async_copy(
                            orows, acc.at[idx_d.at[ch - 1]], ossem).wait()
                    pltpu.async_copy(y_hbm.at[idx_s.at[ch + 1]], orows, ogsem)

                pltpu.make_async_copy(y_hbm.at[idx_s.at[ch]], rows, gsem).wait()
                pltpu.async_copy(rows, acc.at[idx_d.at[ch]], ssem, add=True)
            return carry

        lax.fori_loop(0, HALF_ROWS // 2, _pair, 0)
        pltpu.make_async_copy(rows0, acc.at[idx_d.at[HALF_ROWS - 2]],
                              ssem0).wait()
        pltpu.make_async_copy(rows1, acc.at[idx_d.at[HALF_ROWS - 1]],
                              ssem1).wait()

    plsc.subcore_barrier()

    @pl.when(s < 15)
    def _():
        pltpu.sync_copy(acc.at[pl.ds(s * 624, 624)],
                        part_hbm.at[c, pl.ds(s * 624, 624)])

    @pl.when(s == 15)
    def _():
        pltpu.sync_copy(acc.at[pl.ds(15 * 624, 640)],
                        part_hbm.at[c, pl.ds(15 * 624, 640)])


_sc_agg = functools.partial(
    pl.kernel,
    out_type=jax.ShapeDtypeStruct((NCORE, N, HID), jnp.float32),
    mesh=plsc.VectorSubcoreMesh(core_axis_name="c", subcore_axis_name="s"),
    scratch_types=[
        pltpu.VMEM((HALF_ROWS, CH), jnp.int32),
        pltpu.VMEM((HALF_ROWS, CH), jnp.int32),
        pltpu.VMEM((CH, HID), jnp.float32),
        pltpu.VMEM((CH, HID), jnp.float32),
        pltpu.VMEM((ZROWS, HID), jnp.float32),
        pltpu.VMEM_SHARED((N, HID), jnp.float32),
        pltpu.SemaphoreType.DMA,
        pltpu.SemaphoreType.DMA,
        pltpu.SemaphoreType.DMA,
        pltpu.SemaphoreType.DMA,
    ],
)(_sc_agg_body)


def _k1a_body(x_ref, top_ref, w1a_ref, w1b_ref, xw_ref):
    xw_ref[...] = (jnp.dot(x_ref[...], w1a_ref[...], precision=_P)
                   + jnp.dot(top_ref[...], w1b_ref[...], precision=_P))


def _k1a(x, top, w1a, w1b):
    return pl.pallas_call(
        _k1a_body,
        grid=(GRID,),
        in_specs=[
            pl.BlockSpec((ROWB, NF), lambda i: (i, 0)),
            pl.BlockSpec((ROWB, TF), lambda i: (i, 0)),
            pl.BlockSpec((NF, HID), lambda i: (0, 0)),
            pl.BlockSpec((TF, HID), lambda i: (0, 0)),
        ],
        out_specs=pl.BlockSpec((ROWB, HID), lambda i: (i, 0)),
        out_shape=jax.ShapeDtypeStruct((N, HID), jnp.float32),
    )(x, top, w1a, w1b)


def _k1b_body(xw_ref, p0_ref, p1_ref, y_ref, dinv_ref):
    deg = p0_ref[...] + p1_ref[...] + 1.0
    dinv = lax.rsqrt(deg)
    y_ref[...] = xw_ref[...] * dinv
    dinv_ref[...] = dinv


def _k1b(xw, p0, p1):
    return pl.pallas_call(
        _k1b_body,
        grid=(GRID,),
        in_specs=[
            pl.BlockSpec((ROWB, HID), lambda i: (i, 0)),
            pl.BlockSpec((ROWB, 1), lambda i: (i, 0)),
            pl.BlockSpec((ROWB, 1), lambda i: (i, 0)),
        ],
        out_specs=[
            pl.BlockSpec((ROWB, HID), lambda i: (i, 0)),
            pl.BlockSpec((ROWB, 1), lambda i: (i, 0)),
        ],
        out_shape=[
            jax.ShapeDtypeStruct((N, HID), jnp.float32),
            jax.ShapeDtypeStruct((N, 1), jnp.float32),
        ],
    )(xw, p0, p1)


def _k2_body(pa_ref, pb_ref, y1_ref, dinv_ref, b1_ref, w2_ref, y2_ref):
    dinv = dinv_ref[...]
    h1 = jax.nn.relu((pa_ref[...] + pb_ref[...] + y1_ref[...]) * dinv
                     + b1_ref[...])
    y2_ref[...] = jnp.dot(h1, w2_ref[...], precision=_P) * dinv


def _k2(pa, pb, y1, dinv, b1, w2):
    return pl.pallas_call(
        _k2_body,
        grid=(GRID,),
        in_specs=[
            pl.BlockSpec((ROWB, HID), lambda i: (i, 0)),
            pl.BlockSpec((ROWB, HID), lambda i: (i, 0)),
            pl.BlockSpec((ROWB, HID), lambda i: (i, 0)),
            pl.BlockSpec((ROWB, 1), lambda i: (i, 0)),
            pl.BlockSpec((1, HID), lambda i: (0, 0)),
            pl.BlockSpec((HID, HID), lambda i: (0, 0)),
        ],
        out_specs=pl.BlockSpec((ROWB, HID), lambda i: (i, 0)),
        out_shape=jax.ShapeDtypeStruct((N, HID), jnp.float32),
    )(pa, pb, y1, dinv, b1, w2)


def _k3_body(pa_ref, pb_ref, y2_ref, dinv_ref, b2_ref, batch_ref,
             counts_ref, fcw_ref, fcb_ref, out_ref, pooled):
    i = pl.program_id(0)

    @pl.when(i == 0)
    def _():
        pooled[...] = jnp.zeros((G, HID), jnp.float32)

    h2 = jax.nn.relu((pa_ref[...] + pb_ref[...] + y2_ref[...])
                     * dinv_ref[...] + b2_ref[...])
    bb = batch_ref[0]
    gid = lax.broadcasted_iota(jnp.int32, (G, ROWB), 0)
    mask = (bb == gid).astype(jnp.float32)
    pooled[...] += jnp.dot(mask, h2, precision=_P)

    @pl.when(i == GRID - 1)
    def _():
        cnt = jnp.maximum(counts_ref[...], 1.0)
        out_ref[...] = (jnp.dot(pooled[...] / cnt, fcw_ref[...],
                                precision=_P) + fcb_ref[...])


def _k3(pa, pb, y2, dinv, b2, batch3, counts, fcw, fcb):
    return pl.pallas_call(
        _k3_body,
        grid=(GRID,),
        in_specs=[
            pl.BlockSpec((ROWB, HID), lambda i: (i, 0)),
            pl.BlockSpec((ROWB, HID), lambda i: (i, 0)),
            pl.BlockSpec((ROWB, HID), lambda i: (i, 0)),
            pl.BlockSpec((ROWB, 1), lambda i: (i, 0)),
            pl.BlockSpec((1, HID), lambda i: (0, 0)),
            pl.BlockSpec((1, 1, ROWB), lambda i: (i, 0, 0)),
            pl.BlockSpec((G, 1), lambda i: (0, 0)),
            pl.BlockSpec((HID, OUT), lambda i: (0, 0)),
            pl.BlockSpec((1, OUT), lambda i: (0, 0)),
        ],
        out_specs=pl.BlockSpec((G, OUT), lambda i: (0, 0)),
        out_shape=jax.ShapeDtypeStruct((G, OUT), jnp.float32),
        scratch_shapes=[pltpu.VMEM((G, HID), jnp.float32)],
    )(pa, pb, y2, dinv, b2, batch3, counts, fcw, fcb)


def kernel(x, top_features, edge_index, batch, W1, b1, W2, b2, fc_W, fc_b):
    src2d = edge_index[0].reshape(ECH, CH)
    dst2d = edge_index[1].reshape(ECH, CH)
    pdeg0, pdeg1, counts = _sc_deg(edge_index[1].reshape(NW, 1, EPT),
                                   batch.reshape(NSUB, 1, N // NSUB))
    p0 = pdeg0[:N].reshape(N, 1)
    p1 = pdeg1[:N].reshape(N, 1)

    xw1 = _k1a(x, top_features, W1[:NF], W1[NF:])
    y1, dinv = _k1b(xw1, p0, p1)

    part1 = _sc_agg(y1, src2d, dst2d)
    y2 = _k2(part1[0], part1[1], y1, dinv, b1.reshape(1, HID), W2)

    part2 = _sc_agg(y2, src2d, dst2d)
    out = _k3(part2[0], part2[1], y2, dinv, b2.reshape(1, HID),
              batch.reshape(GRID, 1, ROWB), counts[:G].reshape(G, 1),
              fc_W, fc_b.reshape(1, OUT))
    return out
